# Initial kernel scaffold; baseline (speedup 1.0000x reference)
import jax
import jax.numpy as jnp
from jax.experimental import pallas as pl


def kernel(X, I, attn_kernel):
    raise NotImplementedError("write your pallas kernel here")


# speedup vs baseline: 2.6010x; 2.6010x over previous
import jax
import jax.numpy as jnp
from jax.experimental import pallas as pl
from jax.experimental.pallas import tpu as pltpu

N = 100000
F = 128
G = 512
TILE = 1000
GRID = N // TILE


def _body(x_ref, i_ref, a_ref, o_ref, acc_ref, m_ref, d_ref):
    step = pl.program_id(0)

    @pl.when(step == 0)
    def _init():
        m_ref[0, 0] = -jnp.inf
        d_ref[0, 0] = 0.0
        acc_ref[...] = jnp.zeros_like(acc_ref)

    x = x_ref[...]
    logits = jnp.dot(x, a_ref[...], preferred_element_type=jnp.float32)
    m_old = m_ref[0, 0]
    m_new = jnp.maximum(m_old, jnp.max(logits))
    m_ref[0, 0] = m_new
    scale = jnp.exp(m_old - m_new)
    w = jnp.exp(logits - m_new)
    d_ref[0, 0] = d_ref[0, 0] * scale + jnp.sum(w)

    cols = jax.lax.broadcasted_iota(jnp.int32, (TILE, G), 1)
    p = jnp.where(i_ref[...] == cols, w, 0.0)
    contrib = jax.lax.dot_general(
        p, x, (((0,), (0,)), ((), ())), preferred_element_type=jnp.float32)
    acc_ref[...] = acc_ref[...] * scale + contrib

    @pl.when(step == GRID - 1)
    def _finish():
        o_ref[...] = acc_ref[...] / d_ref[0, 0]


def kernel(X, I, attn_kernel):
    I2 = I.astype(jnp.int32).reshape(N, 1)
    return pl.pallas_call(
        _body,
        grid=(GRID,),
        in_specs=[
            pl.BlockSpec((TILE, F), lambda i: (i, 0)),
            pl.BlockSpec((TILE, 1), lambda i: (i, 0)),
            pl.BlockSpec((F, 1), lambda i: (0, 0)),
        ],
        out_specs=pl.BlockSpec((G, F), lambda i: (0, 0)),
        out_shape=jax.ShapeDtypeStruct((G, F), jnp.float32),
        scratch_shapes=[
            pltpu.VMEM((G, F), jnp.float32),
            pltpu.SMEM((1, 1), jnp.float32),
            pltpu.SMEM((1, 1), jnp.float32),
        ],
        compiler_params=pltpu.CompilerParams(
            dimension_semantics=("arbitrary",),
        ),
    )(X, I2, attn_kernel)


# speedup vs baseline: 3.3362x; 1.2827x over previous
import jax
import jax.numpy as jnp
from jax.experimental import pallas as pl
from jax.experimental.pallas import tpu as pltpu

N = 100000
F = 128
G = 512
TILE = 2000
GRID = N // TILE


def _body(x_ref, i_ref, a_ref, o_ref, acc_ref, m_ref, d_ref):
    step = pl.program_id(0)

    @pl.when(step == 0)
    def _init():
        m_ref[0, 0] = -jnp.inf
        d_ref[0, 0] = 0.0
        acc_ref[...] = jnp.zeros_like(acc_ref)

    x = x_ref[...]
    logits = jnp.dot(x, a_ref[...], preferred_element_type=jnp.float32)
    m_old = m_ref[0, 0]
    m_new = jnp.maximum(m_old, jnp.max(logits))
    m_ref[0, 0] = m_new
    scale = jnp.exp(m_old - m_new)
    w = jnp.exp(logits - m_new)
    d_ref[0, 0] = d_ref[0, 0] * scale + jnp.sum(w)

    cols = jax.lax.broadcasted_iota(jnp.int16, (TILE, G), 1)
    p = (i_ref[...].astype(jnp.int16) == cols).astype(jnp.bfloat16)
    t = (w * x).astype(jnp.bfloat16)
    contrib = jax.lax.dot_general(
        p, t, (((0,), (0,)), ((), ())), preferred_element_type=jnp.float32)

    @pl.when(m_new > m_old)
    def _rescale_add():
        acc_ref[...] = acc_ref[...] * scale + contrib

    @pl.when(jnp.logical_not(m_new > m_old))
    def _plain_add():
        acc_ref[...] = acc_ref[...] + contrib

    @pl.when(step == GRID - 1)
    def _finish():
        o_ref[...] = acc_ref[...] / d_ref[0, 0]


def kernel(X, I, attn_kernel):
    I2 = I.astype(jnp.int32).reshape(N, 1)
    return pl.pallas_call(
        _body,
        grid=(GRID,),
        in_specs=[
            pl.BlockSpec((TILE, F), lambda i: (i, 0)),
            pl.BlockSpec((TILE, 1), lambda i: (i, 0)),
            pl.BlockSpec((F, 1), lambda i: (0, 0)),
        ],
        out_specs=pl.BlockSpec((G, F), lambda i: (0, 0)),
        out_shape=jax.ShapeDtypeStruct((G, F), jnp.float32),
        scratch_shapes=[
            pltpu.VMEM((G, F), jnp.float32),
            pltpu.SMEM((1, 1), jnp.float32),
            pltpu.SMEM((1, 1), jnp.float32),
        ],
        compiler_params=pltpu.CompilerParams(
            dimension_semantics=("arbitrary",),
        ),
    )(X, I2, attn_kernel)
